# baseline (speedup 1.0000x reference)
import numpy as np

import jax
import jax.numpy as jnp
from jax.experimental import pallas as pl

_BB = 16


def _fused_kernel(p0_ref, p1_ref, p2_ref, p3_ref, w1_ref, b1_ref, w2_ref, b2_ref, g1w_ref, g1b_ref,
                  g2w_ref, g2b_ref, fcw_ref, fcb_ref, out_ref):
    tq = p0_ref.shape[2]
    bb = out_ref.shape[0]
    lb = bb * tq

    li = jax.lax.broadcasted_iota(jnp.int32, (1, lb), 1) % tq
    first = li == 0
    last = li == tq - 1

    def shifts(a):
        z = jnp.zeros_like(a[:, :1])
        plus = jnp.where(last, 0.0, jnp.concatenate([a[:, 1:], z], 1))
        minus = jnp.where(first, 0.0, jnp.concatenate([z, a[:, :-1]], 1))
        return minus, plus

    def mm(a, b):
        return jax.lax.dot_general(a, b, (((1,), (0,)), ((), ())),
                                   preferred_element_type=jnp.float32)

    ph = (p0_ref[...], p1_ref[...], p2_ref[...], p3_ref[...])
    xb = jnp.concatenate(
        [jnp.concatenate([p[s] for p in ph], axis=0)
         for s in range(bb)], axis=1)

    m1, p1 = shifts(xb)
    h = jnp.maximum(mm(w1_ref[...], jnp.concatenate([m1, xb, p1], 0))
                    + b1_ref[...], 0.0)
    pe = jnp.maximum(h[0:16], h[16:32])
    po = jnp.maximum(h[32:48], h[48:64])
    h1 = jnp.concatenate([pe, po], axis=0)

    m2, p2 = shifts(h1)
    h = jnp.maximum(mm(w2_ref[...], jnp.concatenate([m2, h1, p2], 0))
                    + b2_ref[...], 0.0)
    nodes = jnp.maximum(h[0:32], h[32:64])

    dinv = jnp.where(first | last, jax.lax.rsqrt(2.0), jax.lax.rsqrt(3.0))

    def gcn(n, w, b):
        g = mm(w, n) * dinv
        gm, gp = shifts(g)
        return jnp.maximum((g + gm + gp) * dinv + b, 0.0)

    nodes = gcn(nodes, g1w_ref[...], g1b_ref[...])
    nodes = gcn(nodes, g2w_ref[...], g2b_ref[...])

    pooled = jnp.sum(nodes.reshape(64, lb // tq, tq), axis=2) * (1.0 / tq)
    out = mm(fcw_ref[...], pooled) + fcb_ref[...]
    out_ref[...] = out.T


@jax.jit
def kernel(x, conv1_w, conv1_b, conv2_w, conv2_b, gcn1_w, gcn1_b, gcn2_w,
           gcn2_b, fc_w, fc_b):
    b, c_in, t_in = x.shape
    tq = t_in // 4
    out_f = fc_w.shape[1]


    w1b = jnp.zeros((64, 108), jnp.float32)
    for p_out in range(4):
        rows = np.arange(16) + 16 * p_out
        for k in range(5):
            r = p_out + k - 2
            cols = (r // 4 + 1) * 36 + (r % 4) * c_in + np.arange(c_in)
            w1b = w1b.at[rows[:, None], cols[None, :]].set(conv1_w[:, :, k])
    w2b = jnp.zeros((64, 96), jnp.float32)
    for j in range(2):
        rows = np.arange(32) + 32 * j
        for k in range(5):
            r = j + k - 2
            cols = (r // 2 + 1) * 32 + (r % 2) * 16 + np.arange(16)
            w2b = w2b.at[rows[:, None], cols[None, :]].set(conv2_w[:, :, k])

    phases = tuple(
        jax.lax.slice(x, (0, 0, p), (b, c_in, t_in), (1, 1, 4))
        for p in range(4))
    args = phases + (w1b, jnp.tile(conv1_b, 4)[:, None],
            w2b, jnp.tile(conv2_b, 2)[:, None],
            gcn1_w.T, gcn1_b[:, None], gcn2_w.T, gcn2_b[:, None],
            fc_w.T, fc_b[:, None])

    full = lambda a: pl.BlockSpec(a.shape, lambda i: (0,) * a.ndim)
    in_specs = [pl.BlockSpec((_BB, c_in, tq), lambda i: (i, 0, 0))
                for _ in range(4)]
    in_specs += [full(a) for a in args[4:]]
    return pl.pallas_call(
        _fused_kernel,
        grid=(b // _BB,),
        in_specs=in_specs,
        out_specs=pl.BlockSpec((_BB, out_f), lambda i: (i, 0)),
        out_shape=jax.ShapeDtypeStruct((b, out_f), x.dtype),
    )(*args)


# speedup vs baseline: 3.2648x; 3.2648x over previous
import numpy as np

import jax
import jax.numpy as jnp
from jax.experimental import pallas as pl

_BB = 16


def _fused_kernel(x_ref, w1_ref, b1_ref, w2_ref, b2_ref, g1w_ref, g1b_ref,
                  g2w_ref, g2b_ref, fcw_ref, fcb_ref, out_ref):
    tq = x_ref.shape[1]
    bb = out_ref.shape[0]
    lb = bb * tq

    li = jax.lax.broadcasted_iota(jnp.int32, (1, lb), 1) % tq
    first = li == 0
    last = li == tq - 1

    def shifts(a):
        z = jnp.zeros_like(a[:, :1])
        plus = jnp.where(last, 0.0, jnp.concatenate([a[:, 1:], z], 1))
        minus = jnp.where(first, 0.0, jnp.concatenate([z, a[:, :-1]], 1))
        return minus, plus

    def mm(a, b):
        return jax.lax.dot_general(a, b, (((1,), (0,)), ((), ())),
                                   preferred_element_type=jnp.float32)

    xr = x_ref[...]
    nrow = xr.shape[0] // bb
    xb = jnp.concatenate(
        [xr[s * nrow:(s + 1) * nrow, :] for s in range(bb)], axis=1)

    m1, p1 = shifts(xb)
    h = jnp.maximum(mm(w1_ref[...], jnp.concatenate([m1, xb, p1], 0))
                    + b1_ref[...], 0.0)
    pe = jnp.maximum(h[0:16], h[16:32])
    po = jnp.maximum(h[32:48], h[48:64])
    h1 = jnp.concatenate([pe, po], axis=0)

    m2, p2 = shifts(h1)
    h = jnp.maximum(mm(w2_ref[...], jnp.concatenate([m2, h1, p2], 0))
                    + b2_ref[...], 0.0)
    nodes = jnp.maximum(h[0:32], h[32:64])

    dinv = jnp.where(first | last, jax.lax.rsqrt(2.0), jax.lax.rsqrt(3.0))

    def gcn(n, w, b):
        g = mm(w, n) * dinv
        gm, gp = shifts(g)
        return jnp.maximum((g + gm + gp) * dinv + b, 0.0)

    nodes = gcn(nodes, g1w_ref[...], g1b_ref[...])
    nodes = gcn(nodes, g2w_ref[...], g2b_ref[...])

    pooled = jnp.sum(nodes.reshape(64, lb // tq, tq), axis=2) * (1.0 / tq)
    out = mm(fcw_ref[...], pooled) + fcb_ref[...]
    out_ref[...] = out.T


@jax.jit
def kernel(x, conv1_w, conv1_b, conv2_w, conv2_b, gcn1_w, gcn1_b, gcn2_w,
           gcn2_b, fc_w, fc_b):
    b, c_in, t_in = x.shape
    tq = t_in // 4
    out_f = fc_w.shape[1]

    w_id = np.zeros((4 * c_in, c_in, 4), np.float32)
    for c in range(c_in):
        for p in range(4):
            w_id[c * 4 + p, c, p] = 1.0
    xr = jax.lax.conv_general_dilated(
        x, jnp.asarray(w_id), window_strides=(4,), padding=[(0, 0)],
        dimension_numbers=('NCH', 'OIH', 'NCH')).reshape(b * c_in * 4, tq)

    w1b = jnp.zeros((64, 108), jnp.float32)
    for p_out in range(4):
        rows = np.arange(16) + 16 * p_out
        for k in range(5):
            r = p_out + k - 2
            cols = (r // 4 + 1) * 36 + np.arange(c_in) * 4 + r % 4
            w1b = w1b.at[rows[:, None], cols[None, :]].set(conv1_w[:, :, k])
    w2b = jnp.zeros((64, 96), jnp.float32)
    for j in range(2):
        rows = np.arange(32) + 32 * j
        for k in range(5):
            r = j + k - 2
            cols = (r // 2 + 1) * 32 + (r % 2) * 16 + np.arange(16)
            w2b = w2b.at[rows[:, None], cols[None, :]].set(conv2_w[:, :, k])

    args = (xr, w1b, jnp.tile(conv1_b, 4)[:, None],
            w2b, jnp.tile(conv2_b, 2)[:, None],
            gcn1_w.T, gcn1_b[:, None], gcn2_w.T, gcn2_b[:, None],
            fc_w.T, fc_b[:, None])

    full = lambda a: pl.BlockSpec(a.shape, lambda i: (0,) * a.ndim)
    in_specs = [pl.BlockSpec((_BB * c_in * 4, tq), lambda i: (i, 0))]
    in_specs += [full(a) for a in args[1:]]
    return pl.pallas_call(
        _fused_kernel,
        grid=(b // _BB,),
        in_specs=in_specs,
        out_specs=pl.BlockSpec((_BB, out_f), lambda i: (i, 0)),
        out_shape=jax.ShapeDtypeStruct((b, out_f), x.dtype),
    )(*args)


# speedup vs baseline: 4.8226x; 1.4771x over previous
import numpy as np

import jax
import jax.numpy as jnp
from jax.experimental import pallas as pl

_BB = 16


def _fused_kernel(x_ref, w1_ref, b1_ref, w2_ref, b2_ref, g1w_ref, g1b_ref,
                  g2w_ref, g2b_ref, fcw_ref, fcb_ref, out_ref):
    tq = x_ref.shape[1]
    bb = out_ref.shape[0]
    lb = bb * tq

    li = jax.lax.broadcasted_iota(jnp.int32, (1, lb), 1) % tq
    first = li == 0
    last = li == tq - 1

    def shifts(a):
        z = jnp.zeros_like(a[:, :1])
        plus = jnp.where(last, 0.0, jnp.concatenate([a[:, 1:], z], 1))
        minus = jnp.where(first, 0.0, jnp.concatenate([z, a[:, :-1]], 1))
        return minus, plus

    def mm(a, b):
        return jax.lax.dot_general(a, b, (((1,), (0,)), ((), ())),
                                   preferred_element_type=jnp.float32)

    xr = x_ref[...]
    nrow = xr.shape[0] // bb
    xb = jnp.concatenate(
        [xr[s * nrow:(s + 1) * nrow, :] for s in range(bb)], axis=1)

    m1, p1 = shifts(xb)
    h = jnp.maximum(mm(w1_ref[...], jnp.concatenate([m1, xb, p1], 0))
                    + b1_ref[...], 0.0)
    pe = jnp.maximum(h[0:16], h[16:32])
    po = jnp.maximum(h[32:48], h[48:64])
    h1 = jnp.concatenate([pe, po], axis=0)

    m2, p2 = shifts(h1)
    h = jnp.maximum(mm(w2_ref[...], jnp.concatenate([m2, h1, p2], 0))
                    + b2_ref[...], 0.0)
    nodes = jnp.maximum(h[0:32], h[32:64])

    dinv = jnp.where(first | last, jax.lax.rsqrt(2.0), jax.lax.rsqrt(3.0))

    def gcn(n, w, b):
        g = mm(w, n) * dinv
        gm, gp = shifts(g)
        return jnp.maximum((g + gm + gp) * dinv + b, 0.0)

    nodes = gcn(nodes, g1w_ref[...], g1b_ref[...])
    nodes = gcn(nodes, g2w_ref[...], g2b_ref[...])

    pooled = jnp.sum(nodes.reshape(64, lb // tq, tq), axis=2) * (1.0 / tq)
    out = mm(fcw_ref[...], pooled) + fcb_ref[...]
    out_ref[...] = out.T


@jax.jit
def kernel(x, conv1_w, conv1_b, conv2_w, conv2_b, gcn1_w, gcn1_b, gcn2_w,
           gcn2_b, fc_w, fc_b):
    b, c_in, t_in = x.shape
    tq = t_in // 4
    out_f = fc_w.shape[1]

    xr = x.reshape(b, c_in, tq, 4).transpose(0, 1, 3, 2).reshape(
        b * c_in * 4, tq)

    idx1 = np.full((64, 108), 16 * 9 * 5, np.int64)
    for p_out in range(4):
        for k in range(5):
            r = p_out + k - 2
            cols = (r // 4 + 1) * 36 + np.arange(c_in) * 4 + r % 4
            rows = np.arange(16) + 16 * p_out
            idx1[rows[:, None], cols[None, :]] = (
                (np.arange(16)[:, None] * 9 + np.arange(9)[None, :]) * 5 + k)
    idx2 = np.full((64, 96), 32 * 16 * 5, np.int64)
    for j in range(2):
        for k in range(5):
            r = j + k - 2
            cols = (r // 2 + 1) * 32 + (r % 2) * 16 + np.arange(16)
            rows = np.arange(32) + 32 * j
            idx2[rows[:, None], cols[None, :]] = (
                (np.arange(32)[:, None] * 16 + np.arange(16)[None, :]) * 5 + k)
    w1b = jnp.concatenate([conv1_w.reshape(-1), jnp.zeros(1, jnp.float32)])[idx1]
    w2b = jnp.concatenate([conv2_w.reshape(-1), jnp.zeros(1, jnp.float32)])[idx2]

    args = (xr, w1b, conv1_b[np.tile(np.arange(16), 4)][:, None],
            w2b, conv2_b[np.tile(np.arange(32), 2)][:, None],
            gcn1_w.T, gcn1_b[:, None], gcn2_w.T, gcn2_b[:, None],
            fc_w.T, fc_b[:, None])

    full = lambda a: pl.BlockSpec(a.shape, lambda i: (0,) * a.ndim)
    in_specs = [pl.BlockSpec((_BB * c_in * 4, tq), lambda i: (i, 0))]
    in_specs += [full(a) for a in args[1:]]
    return pl.pallas_call(
        _fused_kernel,
        grid=(b // _BB,),
        in_specs=in_specs,
        out_specs=pl.BlockSpec((_BB, out_f), lambda i: (i, 0)),
        out_shape=jax.ShapeDtypeStruct((b, out_f), x.dtype),
    )(*args)
